# baseline (speedup 1.0000x reference)
import functools

import jax
import jax.numpy as jnp
import numpy as np
from jax import lax
from jax.experimental import pallas as pl
from jax.experimental.pallas import tpu as pltpu
from jax.experimental.pallas import tpu_sc as plsc

_G = 20000
_SETS = 64
_B = 8
_LANES = 16
_CHUNKS = _G // _LANES


@functools.partial(
    pl.kernel,
    out_type=jax.ShapeDtypeStruct((_B, _G), jnp.float32),
    mesh=plsc.VectorSubcoreMesh(core_axis_name="c", subcore_axis_name="s"),
    scratch_types=[
        pltpu.VMEM((_G,), jnp.int32),
        pltpu.VMEM((_G,), jnp.float32),
    ],
    compiler_params=pltpu.CompilerParams(needs_layout_passes=False),
)
def _rank_weights(s_hbm, t_hbm, idx_v, row_v):
    wid = lax.axis_index("s") * 2 + lax.axis_index("c")

    @pl.when(wid < _B)
    def _():
        pltpu.sync_copy(s_hbm.at[wid], idx_v)
        iota = lax.iota(jnp.int32, _LANES)

        @plsc.parallel_loop(0, _CHUNKS, unroll=8)
        def _loop(i):
            base = i * _LANES
            idx = idx_v[pl.ds(base, _LANES)]
            vals = (_G - base) - iota
            plsc.store_scatter(row_v, [idx], vals.astype(jnp.float32))

        pltpu.sync_copy(row_v, t_hbm.at[wid])


_DN = (((1,), (1,)), ((), ()))
_HI = lax.Precision.HIGHEST


def _prep_body(r_ref, sm_ref, rhs_ref, den_ref):
    ind = jax.nn.sigmoid(sm_ref[...])
    avg = jnp.mean(ind, axis=1, keepdims=True)
    ind = jnp.where(ind < avg * 0.3, ind * 0.01, ind)
    ia = jnp.sqrt(jnp.sqrt(ind))
    neg = (ind < 0.1).astype(jnp.float32)
    rhs = jnp.concatenate([ia, neg], axis=0)
    rhs_ref[...] = rhs
    ra = jnp.sqrt(jnp.sqrt(r_ref[...]))
    ones = jnp.ones((_B, _G), jnp.float32)
    lhs = jnp.concatenate([ra, ones], axis=0)
    den_ref[...] = lax.dot_general(lhs, rhs, _DN, precision=_HI,
                                   preferred_element_type=jnp.float32)


_prep_call = pl.pallas_call(
    _prep_body,
    out_shape=[
        jax.ShapeDtypeStruct((2 * _SETS, _G), jnp.float32),
        jax.ShapeDtypeStruct((2 * _B, 2 * _SETS), jnp.float32),
    ],
)


def _fin_body(r_ref, t_ref, rhs_ref, den_ref, out_ref):
    ra = jnp.sqrt(jnp.sqrt(r_ref[...]))
    t = t_ref[...]
    lhs = jnp.concatenate([ra * t, t], axis=0)
    out = lax.dot_general(lhs, rhs_ref[...], _DN, precision=_HI,
                          preferred_element_type=jnp.float32)
    num_pos = out[0:_B, 0:_SETS]
    num_neg = out[_B:2 * _B, _SETS:2 * _SETS]
    den_pos = den_ref[0:_B, 0:_SETS]
    den_neg = den_ref[_B:_B + 1, _SETS:2 * _SETS]
    p = num_pos / (den_pos + 1e-10)
    n = jnp.where(den_neg > 1e-8, num_neg / (den_neg + 1e-10), 0.0)
    out_ref[...] = (p - n) / np.float32(_G)


_fin_call = pl.pallas_call(
    _fin_body,
    out_shape=jax.ShapeDtypeStruct((_B, _SETS), jnp.float32),
)


def kernel(R, S, set_membership):
    t = _rank_weights(S)
    rhs, den = _prep_call(R, set_membership)
    return _fin_call(R, t, rhs, den)


# speedup vs baseline: 1.1944x; 1.1944x over previous
import functools

import jax
import jax.numpy as jnp
import numpy as np
from jax import lax
from jax.experimental import pallas as pl
from jax.experimental.pallas import tpu as pltpu
from jax.experimental.pallas import tpu_sc as plsc

_G = 20000
_SETS = 64
_B = 8
_LANES = 16
_CHUNKS = _G // _LANES


@functools.partial(
    pl.kernel,
    out_type=jax.ShapeDtypeStruct((_B, _G), jnp.float32),
    mesh=plsc.VectorSubcoreMesh(core_axis_name="c", subcore_axis_name="s",
                                num_cores=1),
    scratch_types=[
        pltpu.VMEM((_G,), jnp.int32),
        pltpu.VMEM((_G,), jnp.float32),
    ],
    compiler_params=pltpu.CompilerParams(needs_layout_passes=False),
)
def _rank_weights(s_hbm, t_hbm, idx_v, row_v):
    wid = lax.axis_index("s") + lax.axis_index("c")

    @pl.when(wid < _B)
    def _():
        pltpu.sync_copy(s_hbm.at[wid], idx_v)
        iota = lax.iota(jnp.int32, _LANES)

        @plsc.parallel_loop(0, _CHUNKS, unroll=8)
        def _loop(i):
            base = i * _LANES
            idx = idx_v[pl.ds(base, _LANES)]
            vals = (_G - base) - iota
            plsc.store_scatter(row_v, [idx], vals.astype(jnp.float32))

        pltpu.sync_copy(row_v, t_hbm.at[wid])


def _es_body(r_ref, t_ref, sm_ref, out_ref):
    ind = jax.nn.sigmoid(sm_ref[...])
    avg = jnp.mean(ind, axis=1, keepdims=True)
    ind = jnp.where(ind < avg * 0.3, ind * 0.01, ind)
    ia = jnp.sqrt(jnp.sqrt(ind))
    neg = (ind < 0.1).astype(jnp.float32)
    ra = jnp.sqrt(jnp.sqrt(r_ref[...]))
    t = t_ref[...]
    lhs = jnp.concatenate([ra * t, ra, t], axis=0)
    rhs = jnp.concatenate([ia, neg], axis=0)
    dn = (((1,), (1,)), ((), ()))
    out = lax.dot_general(lhs, rhs, dn, precision=lax.Precision.HIGHEST,
                          preferred_element_type=jnp.float32)
    num_pos = out[0:8, 0:64]
    den_pos = out[8:16, 0:64]
    num_neg = out[16:24, 64:128]
    den_neg = jnp.sum(neg, axis=1)[None, :]
    p = num_pos / (den_pos + 1e-10)
    n = jnp.where(den_neg > 1e-8, num_neg / (den_neg + 1e-10), 0.0)
    out_ref[...] = (p - n) / np.float32(_G)


_es_call = pl.pallas_call(
    _es_body,
    out_shape=jax.ShapeDtypeStruct((_B, _SETS), jnp.float32),
)


def kernel(R, S, set_membership):
    t = _rank_weights(S)
    return _es_call(R, t, set_membership)


# speedup vs baseline: 1.3245x; 1.1089x over previous
import functools

import jax
import jax.numpy as jnp
import numpy as np
from jax import lax
from jax.experimental import pallas as pl
from jax.experimental.pallas import tpu as pltpu
from jax.experimental.pallas import tpu_sc as plsc

_G = 20000
_SETS = 64
_B = 8
_LANES = 16
_CHUNKS = _G // _LANES


@functools.partial(
    pl.kernel,
    out_type=jax.ShapeDtypeStruct((_B, _G), jnp.float32),
    mesh=plsc.VectorSubcoreMesh(core_axis_name="c", subcore_axis_name="s",
                                num_cores=1),
    scratch_types=[
        pltpu.VMEM((_G,), jnp.int32),
        pltpu.VMEM((_G,), jnp.float32),
    ],
    compiler_params=pltpu.CompilerParams(needs_layout_passes=False),
)
def _rank_weights(s_hbm, t_hbm, idx_v, row_v):
    wid = lax.axis_index("s") + lax.axis_index("c")

    @pl.when(wid < _B)
    def _():
        pltpu.sync_copy(s_hbm.at[wid], idx_v)
        iota = lax.iota(jnp.int32, _LANES)

        @plsc.parallel_loop(0, _CHUNKS, unroll=8)
        def _loop(i):
            base = i * _LANES
            idx = idx_v[pl.ds(base, _LANES)]
            vals = (_G - base) - iota
            plsc.store_scatter(row_v, [idx], vals.astype(jnp.float32))

        pltpu.sync_copy(row_v, t_hbm.at[wid])


def _es_body(r_ref, t_ref, sm_ref, out_ref):
    ind = jax.nn.sigmoid(sm_ref[...])
    avg = jnp.mean(ind, axis=1, keepdims=True)
    ind = jnp.where(ind < avg * 0.3, ind * 0.01, ind)
    ia = lax.rsqrt(lax.rsqrt(ind))
    neg = (ind < 0.1).astype(jnp.float32)
    ra = lax.rsqrt(lax.rsqrt(r_ref[...]))
    t = t_ref[...]
    lhs = jnp.concatenate([ra * t, ra, t], axis=0)
    rhs = jnp.concatenate([ia, neg], axis=0)
    dn = (((1,), (1,)), ((), ()))
    lhs_hi = lhs.astype(jnp.bfloat16)
    lhs_lo = (lhs - lhs_hi.astype(jnp.float32)).astype(jnp.bfloat16)
    rhs_hi = rhs.astype(jnp.bfloat16)
    rhs_lo = (rhs - rhs_hi.astype(jnp.float32)).astype(jnp.bfloat16)
    out = (lax.dot_general(lhs_hi, rhs_hi, dn, preferred_element_type=jnp.float32)
           + lax.dot_general(lhs_lo, rhs_hi, dn, preferred_element_type=jnp.float32)
           + lax.dot_general(lhs_hi, rhs_lo, dn, preferred_element_type=jnp.float32))
    num_pos = out[0:8, 0:64]
    den_pos = out[8:16, 0:64]
    num_neg = out[16:24, 64:128]
    den_neg = jnp.sum(neg, axis=1)[None, :]
    p = num_pos / (den_pos + 1e-10)
    n = jnp.where(den_neg > 1e-8, num_neg / (den_neg + 1e-10), 0.0)
    out_ref[...] = (p - n) / np.float32(_G)


_es_call = pl.pallas_call(
    _es_body,
    out_shape=jax.ShapeDtypeStruct((_B, _SETS), jnp.float32),
)


def kernel(R, S, set_membership):
    t = _rank_weights(S)
    return _es_call(R, t, set_membership)


# speedup vs baseline: 1.3732x; 1.0368x over previous
import functools

import jax
import jax.numpy as jnp
import numpy as np
from jax import lax
from jax.experimental import pallas as pl
from jax.experimental.pallas import tpu as pltpu
from jax.experimental.pallas import tpu_sc as plsc

_G = 20000
_SETS = 64
_B = 8
_LANES = 16
_CHUNKS = _G // _LANES


@functools.partial(
    pl.kernel,
    out_type=jax.ShapeDtypeStruct((_B, _G), jnp.float32),
    mesh=plsc.VectorSubcoreMesh(core_axis_name="c", subcore_axis_name="s",
                                num_cores=1),
    scratch_types=[
        pltpu.VMEM((_G,), jnp.int32),
        pltpu.VMEM((_G,), jnp.float32),
    ],
    compiler_params=pltpu.CompilerParams(needs_layout_passes=False),
)
def _rank_weights(s_hbm, t_hbm, idx_v, row_v):
    wid = lax.axis_index("s") + lax.axis_index("c")

    @pl.when(wid < _B)
    def _():
        pltpu.sync_copy(s_hbm.at[wid], idx_v)
        iota = lax.iota(jnp.int32, _LANES)

        @plsc.parallel_loop(0, _CHUNKS, unroll=16)
        def _loop(i):
            base = i * _LANES
            idx = idx_v[pl.ds(base, _LANES)]
            vals = (_G - base) - iota
            plsc.store_scatter(row_v, [idx], vals.astype(jnp.float32))

        pltpu.sync_copy(row_v, t_hbm.at[wid])


def _es_body(r_ref, t_ref, sm_ref, out_ref):
    ind = jax.nn.sigmoid(sm_ref[...])
    avg = jnp.mean(ind, axis=1, keepdims=True)
    ind = jnp.where(ind < avg * 0.3, ind * 0.01, ind)
    ia = lax.rsqrt(lax.rsqrt(ind))
    neg = (ind < 0.1).astype(jnp.float32)
    ra = lax.rsqrt(lax.rsqrt(r_ref[...]))
    t = t_ref[...]
    lhs = jnp.concatenate([ra * t, ra, t], axis=0)
    dn = (((1,), (1,)), ((), ()))
    lhs_hi = lhs.astype(jnp.bfloat16)
    lhs_lo = (lhs - lhs_hi.astype(jnp.float32)).astype(jnp.bfloat16)
    ia_hi = ia.astype(jnp.bfloat16)
    ia_lo = (ia - ia_hi.astype(jnp.float32)).astype(jnp.bfloat16)
    rhs_hi = jnp.concatenate([ia_hi, neg.astype(jnp.bfloat16)], axis=0)
    rhs_lo = jnp.concatenate(
        [ia_lo, jnp.zeros((_SETS, _G), jnp.bfloat16)], axis=0)
    out = (lax.dot_general(lhs_hi, rhs_hi, dn, preferred_element_type=jnp.float32)
           + lax.dot_general(lhs_lo, rhs_hi, dn, preferred_element_type=jnp.float32)
           + lax.dot_general(lhs_hi, rhs_lo, dn, preferred_element_type=jnp.float32))
    num_pos = out[0:8, 0:64]
    den_pos = out[8:16, 0:64]
    num_neg = out[16:24, 64:128]
    den_neg = jnp.sum(neg, axis=1)[None, :]
    p = num_pos / (den_pos + 1e-10)
    n = jnp.where(den_neg > 1e-8, num_neg / (den_neg + 1e-10), 0.0)
    out_ref[...] = (p - n) / np.float32(_G)


_es_call = pl.pallas_call(
    _es_body,
    out_shape=jax.ShapeDtypeStruct((_B, _SETS), jnp.float32),
)


def kernel(R, S, set_membership):
    t = _rank_weights(S)
    return _es_call(R, t, set_membership)


# speedup vs baseline: 1.4482x; 1.0546x over previous
import functools

import jax
import jax.numpy as jnp
import numpy as np
from jax import lax
from jax.experimental import pallas as pl
from jax.experimental.pallas import tpu as pltpu
from jax.experimental.pallas import tpu_sc as plsc

_G = 20000
_SETS = 64
_B = 8
_LANES = 16
_CHUNKS = _G // _LANES


@functools.partial(
    pl.kernel,
    out_type=jax.ShapeDtypeStruct((_B, _G), jnp.float32),
    mesh=plsc.VectorSubcoreMesh(core_axis_name="c", subcore_axis_name="s",
                                num_cores=1),
    scratch_types=[
        pltpu.VMEM((_G,), jnp.int32),
        pltpu.VMEM((_G,), jnp.float32),
    ],
    compiler_params=pltpu.CompilerParams(needs_layout_passes=False),
)
def _rank_weights(s_hbm, t_hbm, idx_v, row_v):
    wid = lax.axis_index("s") + lax.axis_index("c")

    @pl.when(wid < _B)
    def _():
        pltpu.sync_copy(s_hbm.at[wid], idx_v)
        iota = lax.iota(jnp.int32, _LANES)

        @plsc.parallel_loop(0, _CHUNKS, unroll=16)
        def _loop(i):
            base = i * _LANES
            idx = idx_v[pl.ds(base, _LANES)]
            vals = (_G - base) - iota
            plsc.store_scatter(row_v, [idx], vals.astype(jnp.float32))

        pltpu.sync_copy(row_v, t_hbm.at[wid])


def _es_body(r_ref, t_ref, sm_ref, out_ref):
    ind = jax.nn.sigmoid(sm_ref[...])
    avg = jnp.mean(ind, axis=1, keepdims=True)
    ind = jnp.where(ind < avg * 0.3, ind * 0.01, ind)
    ia = lax.rsqrt(lax.rsqrt(ind))
    neg = (ind < 0.1).astype(jnp.float32)
    ra = lax.rsqrt(lax.rsqrt(r_ref[...]))
    t = t_ref[...]
    lhs = jnp.concatenate([ra * t, ra, t], axis=0)
    dn = (((1,), (1,)), ((), ()))
    lhs_hi = lhs.astype(jnp.bfloat16)
    lhs_lo = (lhs - lhs_hi.astype(jnp.float32)).astype(jnp.bfloat16)
    ia_hi = ia.astype(jnp.bfloat16)
    ia_lo = (ia - ia_hi.astype(jnp.float32)).astype(jnp.bfloat16)
    big_lhs = jnp.concatenate([lhs_hi, lhs_lo], axis=0)
    big_rhs = jnp.concatenate([ia_hi, neg.astype(jnp.bfloat16), ia_lo], axis=0)
    res = lax.dot_general(big_lhs, big_rhs, dn,
                          preferred_element_type=jnp.float32)
    sum01 = res[0:24, :] + res[24:48, :]
    num_pos = sum01[0:8, 0:64] + res[0:8, 128:192]
    den_pos = sum01[8:16, 0:64] + res[8:16, 128:192]
    num_neg = sum01[16:24, 64:128]
    den_neg = jnp.sum(neg, axis=1)[None, :]
    p = num_pos / (den_pos + 1e-10)
    n = jnp.where(den_neg > 1e-8, num_neg / (den_neg + 1e-10), 0.0)
    out_ref[...] = (p - n) / np.float32(_G)


_es_call = pl.pallas_call(
    _es_body,
    out_shape=jax.ShapeDtypeStruct((_B, _SETS), jnp.float32),
)


def kernel(R, S, set_membership):
    t = _rank_weights(S)
    return _es_call(R, t, set_membership)
